# baseline (speedup 1.0000x reference)
import math

import jax
import jax.numpy as jnp
from jax.experimental import pallas as pl
from jax.experimental.pallas import tpu as pltpu

_B, _P, _L, _DV, _H, _NH, _NL = 2, 576, 448, 768, 1024, 8, 6
_DF = 4 * _H
_S = _P + _L
_DH = _H // _NH
_EPS = 1e-5
_BF = jnp.bfloat16


def _ln_rows(y, g, b):
    m = jnp.mean(y, axis=-1, keepdims=True)
    c = y - m
    v = jnp.mean(c * c, axis=-1, keepdims=True)
    return c * jax.lax.rsqrt(v + _EPS) * g + b


def _dot_t(a, b):
    return jax.lax.dot_general(
        a, b, (((1,), (1,)), ((), ())), preferred_element_type=jnp.float32)


def _dot(a, b):
    return jax.lax.dot_general(
        a, b, (((1,), (0,)), ((), ())), preferred_element_type=jnp.float32)


def _projln_body(x_ref, w_ref, b_ref, g_ref, bb_ref, o_ref):
    x = x_ref[...].astype(_BF)
    w = w_ref[...].astype(_BF)
    y = _dot_t(x, w) + b_ref[...]
    o_ref[...] = _ln_rows(y, g_ref[...], bb_ref[...])


def _projln(x, w, b, g, beta, tm):
    n, k = x.shape
    h = w.shape[0]
    b2, g2, beta2 = b.reshape(1, h), g.reshape(1, h), beta.reshape(1, h)
    return pl.pallas_call(
        _projln_body,
        grid=(n // tm,),
        in_specs=[
            pl.BlockSpec((tm, k), lambda r: (r, 0)),
            pl.BlockSpec((h, k), lambda r: (0, 0)),
            pl.BlockSpec((1, h), lambda r: (0, 0)),
            pl.BlockSpec((1, h), lambda r: (0, 0)),
            pl.BlockSpec((1, h), lambda r: (0, 0)),
        ],
        out_specs=pl.BlockSpec((tm, h), lambda r: (r, 0)),
        out_shape=jax.ShapeDtypeStruct((n, h), jnp.float32),
    )(x, w, b2, g2, beta2)


def _qkv_body(x_ref, w_ref, b_ref, o_ref):
    x = x_ref[...].astype(_BF)
    w = w_ref[0].astype(_BF)
    y = _dot_t(x, w) + b_ref[0]
    o_ref[...] = y.astype(_BF)


def _qkv_matmul(x2, wqkv, bqkv3, li, tn):
    n = x2.shape[0]
    return pl.pallas_call(
        _qkv_body,
        grid=(3 * _H // tn,),
        in_specs=[
            pl.BlockSpec((n, _H), lambda c: (0, 0)),
            pl.BlockSpec((1, tn, _H), lambda c: (li, c, 0)),
            pl.BlockSpec((1, 1, tn), lambda c: (li, 0, c)),
        ],
        out_specs=pl.BlockSpec((n, tn), lambda c: (0, c)),
        out_shape=jax.ShapeDtypeStruct((n, 3 * _H), _BF),
    )(x2, wqkv, bqkv3)


def _attn_body(q_ref, k_ref, v_ref, wo_ref, bo_ref, x_ref, g_ref, bb_ref,
               o_ref, p_scr, vx_scr, o_scr, wo_scr):
    b = pl.program_id(0)
    h = pl.program_id(1)

    @pl.when(jnp.logical_and(b == 0, h == 0))
    def _():
        vx_scr[:, _DH:] = jnp.ones((_S, _DH), _BF)
        wo_scr[...] = wo_ref[0].astype(_BF)

    s = _dot_t(q_ref[0], k_ref[0]) * (1.0 / math.sqrt(_DH))
    p_scr[...] = jnp.exp(s).astype(_BF)
    vx_scr[:, :_DH] = v_ref[0]
    oe = _dot(p_scr[...], vx_scr[...])
    rs = 1.0 / oe[:, _DH:_DH + 1]
    col = pl.ds(h * _DH, _DH)
    o_scr[:, col] = (oe[:, :_DH] * rs).astype(_BF)

    @pl.when(h == _NH - 1)
    def _():
        proj = _dot_t(o_scr[...], wo_scr[...])
        y = x_ref[0] + proj + bo_ref[0]
        o_ref[0] = _ln_rows(y, g_ref[0], bb_ref[0])


def _attn_block(qkv3, wo, bo3, x, g3, b3, li):
    return pl.pallas_call(
        _attn_body,
        grid=(_B, _NH),
        in_specs=[
            pl.BlockSpec((1, _S, _DH), lambda b, h: (b, 0, h)),
            pl.BlockSpec((1, _S, _DH), lambda b, h: (b, 0, _NH + h)),
            pl.BlockSpec((1, _S, _DH), lambda b, h: (b, 0, 2 * _NH + h)),
            pl.BlockSpec((1, _H, _H), lambda b, h: (li, 0, 0)),
            pl.BlockSpec((1, 1, _H), lambda b, h: (li, 0, 0)),
            pl.BlockSpec((1, _S, _H), lambda b, h: (b, 0, 0)),
            pl.BlockSpec((1, 1, _H), lambda b, h: (li, 0, 0)),
            pl.BlockSpec((1, 1, _H), lambda b, h: (li, 0, 0)),
        ],
        out_specs=pl.BlockSpec((1, _S, _H), lambda b, h: (b, 0, 0)),
        out_shape=jax.ShapeDtypeStruct((_B, _S, _H), jnp.float32),
        scratch_shapes=[
            pltpu.VMEM((_S, _S), _BF),
            pltpu.VMEM((_S, 2 * _DH), _BF),
            pltpu.VMEM((_S, _H), _BF),
            pltpu.VMEM((_H, _H), _BF),
        ],
    )(qkv3, qkv3, qkv3, wo, bo3, x, g3, b3)


def _ffn_body(x_ref, w1_ref, b1_ref, w2_ref, b2_ref, g_ref, bb_ref,
              o_ref, acc_ref, *, ndf):
    d = pl.program_id(0)
    x = x_ref[...]
    xb = x.astype(_BF)
    tdf = w1_ref.shape[1]
    th = tdf // 2
    hps = [_dot_t(xb, w1_ref[0, j * th:(j + 1) * th, :].astype(_BF))
           + b1_ref[0, :, j * th:(j + 1) * th] for j in range(2)]
    has = [0.5 * hp * (1.0 + jax.lax.erf(hp * (1.0 / math.sqrt(2.0))))
           for hp in hps]
    part = (_dot_t(has[0].astype(_BF), w2_ref[0, :, :th].astype(_BF))
            + _dot_t(has[1].astype(_BF), w2_ref[0, :, th:].astype(_BF)))

    @pl.when(d == 0)
    def _():
        acc_ref[...] = part

    @pl.when(d > 0)
    def _():
        acc_ref[...] += part

    @pl.when(d == ndf - 1)
    def _():
        y = x + acc_ref[...] + b2_ref[0]
        o_ref[...] = _ln_rows(y, g_ref[0], bb_ref[0])


def _ffn_block(x, w1, b13, w2, b23, g3, bb3, li, tdf):
    import functools
    n = x.shape[0]
    ndf = _DF // tdf
    body = functools.partial(_ffn_body, ndf=ndf)
    return pl.pallas_call(
        body,
        grid=(ndf,),
        in_specs=[
            pl.BlockSpec((n, _H), lambda d: (0, 0)),
            pl.BlockSpec((1, tdf, _H), lambda d: (li, d, 0)),
            pl.BlockSpec((1, 1, tdf), lambda d: (li, 0, d)),
            pl.BlockSpec((1, _H, tdf), lambda d: (li, 0, d)),
            pl.BlockSpec((1, 1, _H), lambda d: (li, 0, 0)),
            pl.BlockSpec((1, 1, _H), lambda d: (li, 0, 0)),
            pl.BlockSpec((1, 1, _H), lambda d: (li, 0, 0)),
        ],
        out_specs=pl.BlockSpec((n, _H), lambda d: (0, 0)),
        out_shape=jax.ShapeDtypeStruct((n, _H), jnp.float32),
        scratch_shapes=[pltpu.VMEM((n, _H), jnp.float32)],
    )(x, w1, b13, w2, b23, g3, bb3)


def kernel(vision_features, text_features, text_attention_mask, vp_w, vp_b,
           vp_g, vp_beta, tp_w, tp_b, tp_g, tp_beta, Wqkv, bqkv, Wo, bo,
           ln1_g, ln1_b, ve_w1, ve_b1, ve_w2, ve_b2, le_w1, le_b1, le_w2,
           le_b2, ln2_g, ln2_b):
    b = vision_features.shape[0]

    vp = _projln(vision_features.reshape(b * _P, _DV), vp_w, vp_b, vp_g,
                 vp_beta, tm=384)
    tp = _projln(text_features.reshape(b * _L, _H), tp_w, tp_b, tp_g,
                 tp_beta, tm=448)
    x = jnp.concatenate([vp.reshape(b, _P, _H), tp.reshape(b, _L, _H)],
                        axis=1)

    bqkv3 = bqkv.reshape(_NL, 1, 3 * _H)
    bo3 = bo.reshape(_NL, 1, _H)
    g13 = ln1_g.reshape(_NL, 1, _H)
    b13 = ln1_b.reshape(_NL, 1, _H)
    veb13 = ve_b1.reshape(_NL, 1, _DF)
    veb23 = ve_b2.reshape(_NL, 1, _H)
    leb13 = le_b1.reshape(_NL, 1, _DF)
    leb23 = le_b2.reshape(_NL, 1, _H)
    g23 = ln2_g.reshape(_NL, 1, _H)
    b23 = ln2_b.reshape(_NL, 1, _H)

    for li in range(_NL):
        qkv = _qkv_matmul(x.reshape(b * _S, _H), Wqkv, bqkv3, li, tn=1024)
        y = _attn_block(qkv.reshape(b, _S, 3 * _H), Wo, bo3, x, g13, b13, li)
        yv = y[:, :_P].reshape(b * _P, _H)
        yt = y[:, _P:].reshape(b * _L, _H)
        ov = _ffn_block(yv, ve_w1, veb13, ve_w2, veb23, g23, b23, li,
                        tdf=1024)
        ot = _ffn_block(yt, le_w1, leb13, le_w2, leb23, g23, b23, li,
                        tdf=1024)
        x = jnp.concatenate([ov.reshape(b, _P, _H), ot.reshape(b, _L, _H)],
                            axis=1)

    mask = jnp.concatenate(
        [jnp.ones((b, _P), dtype=bool), text_attention_mask.astype(bool)],
        axis=1)
    return x, mask


# speedup vs baseline: 1.0965x; 1.0965x over previous
import functools
import math

import jax
import jax.numpy as jnp
from jax.experimental import pallas as pl
from jax.experimental.pallas import tpu as pltpu

_B, _P, _L, _DV, _H, _NH, _NL = 2, 576, 448, 768, 1024, 8, 6
_DF = 4 * _H
_S = _P + _L
_DH = _H // _NH
_EPS = 1e-5
_BF = jnp.bfloat16


def _ln_rows(y, g, b):
    m = jnp.mean(y, axis=-1, keepdims=True)
    c = y - m
    v = jnp.mean(c * c, axis=-1, keepdims=True)
    return c * jax.lax.rsqrt(v + _EPS) * g + b


def _dot_t(a, b):
    return jax.lax.dot_general(
        a, b, (((1,), (1,)), ((), ())), preferred_element_type=jnp.float32)


def _dot(a, b):
    return jax.lax.dot_general(
        a, b, (((1,), (0,)), ((), ())), preferred_element_type=jnp.float32)


def _projln_body(x_ref, w_ref, b_ref, g_ref, bb_ref, o_ref):
    x = x_ref[...].astype(_BF)
    w = w_ref[...].astype(_BF)
    y = _dot_t(x, w) + b_ref[...]
    o_ref[...] = _ln_rows(y, g_ref[...], bb_ref[...])


def _projln(x, w, b, g, beta, tm):
    n, k = x.shape
    h = w.shape[0]
    b2, g2, beta2 = b.reshape(1, h), g.reshape(1, h), beta.reshape(1, h)
    return pl.pallas_call(
        _projln_body,
        grid=(n // tm,),
        in_specs=[
            pl.BlockSpec((tm, k), lambda r: (r, 0)),
            pl.BlockSpec((h, k), lambda r: (0, 0)),
            pl.BlockSpec((1, h), lambda r: (0, 0)),
            pl.BlockSpec((1, h), lambda r: (0, 0)),
            pl.BlockSpec((1, h), lambda r: (0, 0)),
        ],
        out_specs=pl.BlockSpec((tm, h), lambda r: (r, 0)),
        out_shape=jax.ShapeDtypeStruct((n, h), jnp.float32),
    )(x, w, b2, g2, beta2)


def _qkv_body(x_ref, w_ref, b_ref, o_ref):
    c = pl.program_id(0)
    x = x_ref[...].astype(_BF)
    w = w_ref[0].astype(_BF)
    y = _dot_t(x, w) + b_ref[0]
    f = jnp.where(c == 0, jnp.float32(1.0 / math.sqrt(_DH)), jnp.float32(1.0))
    o_ref[...] = (y * f).astype(_BF)


def _qkv_matmul(x2, wqkv, bqkv3, li):
    n = x2.shape[0]
    tn = _H
    return pl.pallas_call(
        _qkv_body,
        grid=(3 * _H // tn,),
        in_specs=[
            pl.BlockSpec((n, _H), lambda c: (0, 0)),
            pl.BlockSpec((1, tn, _H), lambda c: (li, c, 0)),
            pl.BlockSpec((1, 1, tn), lambda c: (li, 0, c)),
        ],
        out_specs=pl.BlockSpec((n, tn), lambda c: (0, c)),
        out_shape=jax.ShapeDtypeStruct((n, 3 * _H), _BF),
    )(x2, wqkv, bqkv3)


def _attn_body(qv_ref, qt_ref, kv_ref, kt_ref, vv_ref, vt_ref,
               ov_ref, ot_ref, p_scr, vx_scr):
    h = pl.program_id(1)
    vx_scr[:, _DH:] = jnp.ones((_S, _DH), _BF)
    q = jnp.concatenate((qv_ref[0], qt_ref[0]), axis=0)
    k = jnp.concatenate((kv_ref[0], kt_ref[0]), axis=0)
    s = _dot_t(q, k)
    p_scr[...] = jnp.exp(s).astype(_BF)
    vx_scr[:_P, :_DH] = vv_ref[0]
    vx_scr[_P:, :_DH] = vt_ref[0]
    oe = _dot(p_scr[...], vx_scr[...])
    rs = 1.0 / oe[:, _DH:_DH + 1]
    on = (oe[:, :_DH] * rs).astype(_BF)
    col = pl.ds(h * _DH, _DH)
    ov_ref[0, :, col] = on[:_P]
    ot_ref[0, :, col] = on[_P:]


def _attn_block(qv3, qt3, li):
    return pl.pallas_call(
        _attn_body,
        grid=(_B, _NH),
        in_specs=[
            pl.BlockSpec((1, _P, _DH), lambda b, h: (b, 0, h)),
            pl.BlockSpec((1, _L, _DH), lambda b, h: (b, 0, h)),
            pl.BlockSpec((1, _P, _DH), lambda b, h: (b, 0, _NH + h)),
            pl.BlockSpec((1, _L, _DH), lambda b, h: (b, 0, _NH + h)),
            pl.BlockSpec((1, _P, _DH), lambda b, h: (b, 0, 2 * _NH + h)),
            pl.BlockSpec((1, _L, _DH), lambda b, h: (b, 0, 2 * _NH + h)),
        ],
        out_specs=[
            pl.BlockSpec((1, _P, _H), lambda b, h: (b, 0, 0)),
            pl.BlockSpec((1, _L, _H), lambda b, h: (b, 0, 0)),
        ],
        out_shape=[
            jax.ShapeDtypeStruct((_B, _P, _H), _BF),
            jax.ShapeDtypeStruct((_B, _L, _H), _BF),
        ],
        scratch_shapes=[
            pltpu.VMEM((_S, _S), _BF),
            pltpu.VMEM((_S, 2 * _DH), _BF),
        ],
    )(qv3, qt3, qv3, qt3, qv3, qt3)


def _ffn_body(a_ref, z_ref, wo_ref, bo_ref, g1_ref, b1n_ref,
              w1_ref, b1_ref, w2_ref, b2_ref, g2_ref, b2n_ref,
              o_ref, y_scr, acc_ref, *, ndf):
    d = pl.program_id(0)

    @pl.when(d == 0)
    def _():
        proj = _dot_t(a_ref[...], wo_ref[0].astype(_BF))
        y = z_ref[...] + proj + bo_ref[0]
        y_scr[...] = _ln_rows(y, g1_ref[0], b1n_ref[0])

    y = y_scr[...]
    hpre = _dot_t(y.astype(_BF), w1_ref[0].astype(_BF)) + b1_ref[0]
    hact = 0.5 * hpre * (1.0 + jax.lax.erf(hpre * (1.0 / math.sqrt(2.0))))
    part = _dot_t(hact.astype(_BF), w2_ref[0].astype(_BF))

    @pl.when(d == 0)
    def _():
        acc_ref[...] = part

    @pl.when(d > 0)
    def _():
        acc_ref[...] += part

    @pl.when(d == ndf - 1)
    def _():
        o_ref[...] = _ln_rows(y + acc_ref[...] + b2_ref[0],
                              g2_ref[0], b2n_ref[0])


def _ffn_block(a, z, wo, bo3, g13, b13n, w1, b13, w2, b23, g23, bb3, li,
               tdf):
    n = a.shape[0]
    ndf = _DF // tdf
    body = functools.partial(_ffn_body, ndf=ndf)
    return pl.pallas_call(
        body,
        grid=(ndf,),
        in_specs=[
            pl.BlockSpec((n, _H), lambda d: (0, 0)),
            pl.BlockSpec((n, _H), lambda d: (0, 0)),
            pl.BlockSpec((1, _H, _H), lambda d: (li, 0, 0)),
            pl.BlockSpec((1, 1, _H), lambda d: (li, 0, 0)),
            pl.BlockSpec((1, 1, _H), lambda d: (li, 0, 0)),
            pl.BlockSpec((1, 1, _H), lambda d: (li, 0, 0)),
            pl.BlockSpec((1, tdf, _H), lambda d: (li, d, 0)),
            pl.BlockSpec((1, 1, tdf), lambda d: (li, 0, d)),
            pl.BlockSpec((1, _H, tdf), lambda d: (li, 0, d)),
            pl.BlockSpec((1, 1, _H), lambda d: (li, 0, 0)),
            pl.BlockSpec((1, 1, _H), lambda d: (li, 0, 0)),
            pl.BlockSpec((1, 1, _H), lambda d: (li, 0, 0)),
        ],
        out_specs=pl.BlockSpec((n, _H), lambda d: (0, 0)),
        out_shape=jax.ShapeDtypeStruct((n, _H), jnp.float32),
        scratch_shapes=[
            pltpu.VMEM((n, _H), jnp.float32),
            pltpu.VMEM((n, _H), jnp.float32),
        ],
    )(a, z, wo, bo3, g13, b13n, w1, b13, w2, b23, g23, bb3)


def kernel(vision_features, text_features, text_attention_mask, vp_w, vp_b,
           vp_g, vp_beta, tp_w, tp_b, tp_g, tp_beta, Wqkv, bqkv, Wo, bo,
           ln1_g, ln1_b, ve_w1, ve_b1, ve_w2, ve_b2, le_w1, le_b1, le_w2,
           le_b2, ln2_g, ln2_b):
    b = vision_features.shape[0]

    zv = _projln(vision_features.reshape(b * _P, _DV), vp_w, vp_b, vp_g,
                 vp_beta, tm=384)
    zt = _projln(text_features.reshape(b * _L, _H), tp_w, tp_b, tp_g,
                 tp_beta, tm=448)

    bqkv3 = bqkv.reshape(_NL, 1, 3 * _H)
    bo3 = bo.reshape(_NL, 1, _H)
    g13 = ln1_g.reshape(_NL, 1, _H)
    b13n = ln1_b.reshape(_NL, 1, _H)
    veb13 = ve_b1.reshape(_NL, 1, _DF)
    veb23 = ve_b2.reshape(_NL, 1, _H)
    leb13 = le_b1.reshape(_NL, 1, _DF)
    leb23 = le_b2.reshape(_NL, 1, _H)
    g23 = ln2_g.reshape(_NL, 1, _H)
    b23 = ln2_b.reshape(_NL, 1, _H)

    for li in range(_NL):
        qv3 = _qkv_matmul(zv, Wqkv, bqkv3, li)
        qt3 = _qkv_matmul(zt, Wqkv, bqkv3, li)
        av, at = _attn_block(qv3.reshape(b, _P, 3 * _H),
                             qt3.reshape(b, _L, 3 * _H), li)
        zv = _ffn_block(av.reshape(b * _P, _H), zv, Wo, bo3, g13, b13n,
                        ve_w1, veb13, ve_w2, veb23, g23, b23, li, tdf=1024)
        zt = _ffn_block(at.reshape(b * _L, _H), zt, Wo, bo3, g13, b13n,
                        le_w1, leb13, le_w2, leb23, g23, b23, li, tdf=1024)

    x = jnp.concatenate([zv.reshape(b, _P, _H), zt.reshape(b, _L, _H)],
                        axis=1)
    mask = jnp.concatenate(
        [jnp.ones((b, _P), dtype=bool), text_attention_mask.astype(bool)],
        axis=1)
    return x, mask


# speedup vs baseline: 1.1309x; 1.0313x over previous
import functools
import math

import jax
import jax.numpy as jnp
from jax.experimental import pallas as pl
from jax.experimental.pallas import tpu as pltpu

_B, _P, _L, _DV, _H, _NH, _NL = 2, 576, 448, 768, 1024, 8, 6
_DF = 4 * _H
_S = _P + _L
_DH = _H // _NH
_EPS = 1e-5
_BF = jnp.bfloat16


def _ln_rows(y, g, b):
    m = jnp.mean(y, axis=-1, keepdims=True)
    c = y - m
    v = jnp.mean(c * c, axis=-1, keepdims=True)
    return c * jax.lax.rsqrt(v + _EPS) * g + b


def _dot_t(a, b):
    return jax.lax.dot_general(
        a, b, (((1,), (1,)), ((), ())), preferred_element_type=jnp.float32)


def _dot(a, b):
    return jax.lax.dot_general(
        a, b, (((1,), (0,)), ((), ())), preferred_element_type=jnp.float32)


def _projln_body(x_ref, w_ref, b_ref, g_ref, bb_ref, o_ref):
    x = x_ref[...].astype(_BF)
    w = w_ref[...].astype(_BF)
    y = _dot_t(x, w) + b_ref[...]
    o_ref[...] = _ln_rows(y, g_ref[...], bb_ref[...])


def _projln(x, w, b, g, beta, tm):
    n, k = x.shape
    h = w.shape[0]
    b2, g2, beta2 = b.reshape(1, h), g.reshape(1, h), beta.reshape(1, h)
    return pl.pallas_call(
        _projln_body,
        grid=(n // tm,),
        in_specs=[
            pl.BlockSpec((tm, k), lambda r: (r, 0)),
            pl.BlockSpec((h, k), lambda r: (0, 0)),
            pl.BlockSpec((1, h), lambda r: (0, 0)),
            pl.BlockSpec((1, h), lambda r: (0, 0)),
            pl.BlockSpec((1, h), lambda r: (0, 0)),
        ],
        out_specs=pl.BlockSpec((tm, h), lambda r: (r, 0)),
        out_shape=jax.ShapeDtypeStruct((n, h), jnp.float32),
    )(x, w, b2, g2, beta2)


def _qkv_body(x_ref, w_ref, b_ref, o_ref):
    c = pl.program_id(0)
    x = x_ref[...].astype(_BF)
    w = w_ref[0].astype(_BF)
    y = _dot_t(x, w) + b_ref[0]
    f = jnp.where(c == 0, jnp.float32(1.0 / math.sqrt(_DH)), jnp.float32(1.0))
    o_ref[...] = (y * f).astype(_BF)


def _qkv_matmul(x2, wqkv, bqkv3, li):
    n = x2.shape[0]
    tn = _H
    return pl.pallas_call(
        _qkv_body,
        grid=(3 * _H // tn,),
        in_specs=[
            pl.BlockSpec((n, _H), lambda c: (0, 0)),
            pl.BlockSpec((1, tn, _H), lambda c: (li, c, 0)),
            pl.BlockSpec((1, 1, tn), lambda c: (li, 0, c)),
        ],
        out_specs=pl.BlockSpec((n, tn), lambda c: (0, c)),
        out_shape=jax.ShapeDtypeStruct((n, 3 * _H), _BF),
    )(x2, wqkv, bqkv3)


def _attn_body(qv_ref, qt_ref, ov_ref, ot_ref, p_scr, vx_scr):
    vx_scr[:, _DH:] = jnp.ones((_S, _DH), _BF)
    for h in range(_NH):
        hs = pl.ds(h * _DH, _DH)
        ks = pl.ds(_H + h * _DH, _DH)
        vs = pl.ds(2 * _H + h * _DH, _DH)
        q = jnp.concatenate((qv_ref[0, :, hs], qt_ref[0, :, hs]), axis=0)
        k = jnp.concatenate((qv_ref[0, :, ks], qt_ref[0, :, ks]), axis=0)
        s = _dot_t(q, k)
        p_scr[...] = jnp.exp(s).astype(_BF)
        vx_scr[:_P, :_DH] = qv_ref[0, :, vs]
        vx_scr[_P:, :_DH] = qt_ref[0, :, vs]
        oe = _dot(p_scr[...], vx_scr[...])
        rs = 1.0 / oe[:, _DH:_DH + 1]
        on = (oe[:, :_DH] * rs).astype(_BF)
        ov_ref[0, :, hs] = on[:_P]
        ot_ref[0, :, hs] = on[_P:]


def _attn_block(qv3, qt3, li):
    return pl.pallas_call(
        _attn_body,
        grid=(_B,),
        in_specs=[
            pl.BlockSpec((1, _P, 3 * _H), lambda b: (b, 0, 0)),
            pl.BlockSpec((1, _L, 3 * _H), lambda b: (b, 0, 0)),
        ],
        out_specs=[
            pl.BlockSpec((1, _P, _H), lambda b: (b, 0, 0)),
            pl.BlockSpec((1, _L, _H), lambda b: (b, 0, 0)),
        ],
        out_shape=[
            jax.ShapeDtypeStruct((_B, _P, _H), _BF),
            jax.ShapeDtypeStruct((_B, _L, _H), _BF),
        ],
        scratch_shapes=[
            pltpu.VMEM((_S, _S), _BF),
            pltpu.VMEM((_S, 2 * _DH), _BF),
        ],
    )(qv3, qt3)


def _ffn_body(a_ref, z_ref, wo_ref, bo_ref, g1_ref, b1n_ref,
              w1_ref, b1_ref, w2_ref, b2_ref, g2_ref, b2n_ref,
              o_ref, y_scr, acc_ref, *, ndf):
    d = pl.program_id(0)

    @pl.when(d == 0)
    def _():
        proj = _dot_t(a_ref[...], wo_ref[0].astype(_BF))
        y = z_ref[...] + proj + bo_ref[0]
        y_scr[...] = _ln_rows(y, g1_ref[0], b1n_ref[0])

    y = y_scr[...]
    hpre = _dot_t(y.astype(_BF), w1_ref[0].astype(_BF)) + b1_ref[0]
    hact = 0.5 * hpre * (1.0 + jax.lax.erf(hpre * (1.0 / math.sqrt(2.0))))
    part = _dot_t(hact.astype(_BF), w2_ref[0].astype(_BF))

    @pl.when(d == 0)
    def _():
        acc_ref[...] = part

    @pl.when(d > 0)
    def _():
        acc_ref[...] += part

    @pl.when(d == ndf - 1)
    def _():
        o_ref[...] = _ln_rows(y + acc_ref[...] + b2_ref[0],
                              g2_ref[0], b2n_ref[0])


def _ffn_block(a, z, wo, bo3, g13, b13n, w1, b13, w2, b23, g23, bb3, li,
               tdf):
    n = a.shape[0]
    ndf = _DF // tdf
    body = functools.partial(_ffn_body, ndf=ndf)
    return pl.pallas_call(
        body,
        grid=(ndf,),
        in_specs=[
            pl.BlockSpec((n, _H), lambda d: (0, 0)),
            pl.BlockSpec((n, _H), lambda d: (0, 0)),
            pl.BlockSpec((1, _H, _H), lambda d: (li, 0, 0)),
            pl.BlockSpec((1, 1, _H), lambda d: (li, 0, 0)),
            pl.BlockSpec((1, 1, _H), lambda d: (li, 0, 0)),
            pl.BlockSpec((1, 1, _H), lambda d: (li, 0, 0)),
            pl.BlockSpec((1, tdf, _H), lambda d: (li, d, 0)),
            pl.BlockSpec((1, 1, tdf), lambda d: (li, 0, d)),
            pl.BlockSpec((1, _H, tdf), lambda d: (li, 0, d)),
            pl.BlockSpec((1, 1, _H), lambda d: (li, 0, 0)),
            pl.BlockSpec((1, 1, _H), lambda d: (li, 0, 0)),
            pl.BlockSpec((1, 1, _H), lambda d: (li, 0, 0)),
        ],
        out_specs=pl.BlockSpec((n, _H), lambda d: (0, 0)),
        out_shape=jax.ShapeDtypeStruct((n, _H), jnp.float32),
        scratch_shapes=[
            pltpu.VMEM((n, _H), jnp.float32),
            pltpu.VMEM((n, _H), jnp.float32),
        ],
    )(a, z, wo, bo3, g13, b13n, w1, b13, w2, b23, g23, bb3)


def kernel(vision_features, text_features, text_attention_mask, vp_w, vp_b,
           vp_g, vp_beta, tp_w, tp_b, tp_g, tp_beta, Wqkv, bqkv, Wo, bo,
           ln1_g, ln1_b, ve_w1, ve_b1, ve_w2, ve_b2, le_w1, le_b1, le_w2,
           le_b2, ln2_g, ln2_b):
    b = vision_features.shape[0]

    zv = _projln(vision_features.reshape(b * _P, _DV), vp_w, vp_b, vp_g,
                 vp_beta, tm=384)
    zt = _projln(text_features.reshape(b * _L, _H), tp_w, tp_b, tp_g,
                 tp_beta, tm=448)

    bqkv3 = bqkv.reshape(_NL, 1, 3 * _H)
    bo3 = bo.reshape(_NL, 1, _H)
    g13 = ln1_g.reshape(_NL, 1, _H)
    b13n = ln1_b.reshape(_NL, 1, _H)
    veb13 = ve_b1.reshape(_NL, 1, _DF)
    veb23 = ve_b2.reshape(_NL, 1, _H)
    leb13 = le_b1.reshape(_NL, 1, _DF)
    leb23 = le_b2.reshape(_NL, 1, _H)
    g23 = ln2_g.reshape(_NL, 1, _H)
    b23 = ln2_b.reshape(_NL, 1, _H)

    for li in range(_NL):
        qv3 = _qkv_matmul(zv, Wqkv, bqkv3, li)
        qt3 = _qkv_matmul(zt, Wqkv, bqkv3, li)
        av, at = _attn_block(qv3.reshape(b, _P, 3 * _H),
                             qt3.reshape(b, _L, 3 * _H), li)
        zv = _ffn_block(av.reshape(b * _P, _H), zv, Wo, bo3, g13, b13n,
                        ve_w1, veb13, ve_w2, veb23, g23, b23, li, tdf=1024)
        zt = _ffn_block(at.reshape(b * _L, _H), zt, Wo, bo3, g13, b13n,
                        le_w1, leb13, le_w2, leb23, g23, b23, li, tdf=1024)

    x = jnp.concatenate([zv.reshape(b, _P, _H), zt.reshape(b, _L, _H)],
                        axis=1)
    mask = jnp.concatenate(
        [jnp.ones((b, _P), dtype=bool), text_attention_mask.astype(bool)],
        axis=1)
    return x, mask
